# baseline (speedup 1.0000x reference)
import functools

import jax
import jax.numpy as jnp
from jax import lax
from jax.experimental import pallas as pl
from jax.experimental.pallas import tpu as pltpu
from jax.experimental.pallas import tpu_sc as plsc

K = 8192
D = 256
DECAY = 0.99
EPS = 1e-5

TM = 512
N_TOK = 8192
KB = 1024

NC, NS = 2, 16
GCHUNK = 256


def _argmin_body(f_ref, fb_ref, w_ref, wb_ref, idx_ref, wn2_ref):
    i = pl.program_id(0)

    @pl.when(i == 0)
    def _():
        w = w_ref[...]
        wn2_ref[...] = jnp.sum(w * w, axis=1, keepdims=True).reshape(1, K)

    f = f_ref[...]
    fn2 = jnp.sum(f * f, axis=1, keepdims=True)
    mm = lax.dot_general(
        fb_ref[...], wb_ref[...], (((1,), (1,)), ((), ())),
        preferred_element_type=jnp.float32,
    )
    dist = (fn2 + wn2_ref[...]) - 2.0 * mm
    dmin = jnp.min(dist, axis=1, keepdims=True)
    ids = lax.broadcasted_iota(jnp.int32, dist.shape, 1)
    idx = jnp.min(jnp.where(dist == dmin, ids, K), axis=1)
    idx_ref[...] = idx.reshape(1, 1, TM)


def _argmin_tokens(flatten, f_bf16, weight, w_bf16):
    grid = (N_TOK // TM,)
    idx = pl.pallas_call(
        _argmin_body,
        grid=grid,
        in_specs=[
            pl.BlockSpec((TM, D), lambda i: (i, 0)),
            pl.BlockSpec((TM, D), lambda i: (i, 0)),
            pl.BlockSpec((K, D), lambda i: (0, 0)),
            pl.BlockSpec((K, D), lambda i: (0, 0)),
        ],
        out_specs=pl.BlockSpec((1, 1, TM), lambda i: (i, 0, 0)),
        out_shape=jax.ShapeDtypeStruct((grid[0], 1, TM), jnp.int32),
        scratch_shapes=[pltpu.VMEM((1, K), jnp.float32)],
    )(flatten, f_bf16, weight, w_bf16)
    return idx.reshape(-1)


def _segsum_ema_body(idx_ref, fa_ref, csfull_ref, ema_ref, cs_ref, nw_ref,
                     n_ref):
    kb = pl.program_id(0)

    @pl.when(kb == 0)
    def _():
        n_ref[0] = (DECAY * jnp.sum(csfull_ref[...])
                    + (1.0 - DECAY) * N_TOK)

    idx_col = idx_ref[...].reshape(N_TOK, 1)
    ids = kb * KB + lax.broadcasted_iota(jnp.int32, (N_TOK, KB), 1)
    hit = idx_col == ids
    enc = jnp.where(hit, 1.0, 0.0).astype(jnp.float8_e4m3fn)
    emb = lax.dot_general(
        enc, fa_ref[...], (((0,), (0,)), ((), ())),
        preferred_element_type=jnp.float32,
    )
    counts = jnp.sum(jnp.where(hit, 1.0, 0.0), axis=0,
                     keepdims=True).reshape(KB, 1)
    cs = cs_ref[...].reshape(KB, 1) * DECAY + (1.0 - DECAY) * counts
    n = n_ref[0]
    csn = (cs + EPS) / (n + K * EPS) * n
    nw = (ema_ref[...] * DECAY + (1.0 - DECAY) * emb) / csn
    nw_ref[...] = nw.astype(jnp.bfloat16).astype(jnp.float32)


def _segsum_ema(f_bf16, idx, ema_w, cluster_size):
    return pl.pallas_call(
        _segsum_ema_body,
        grid=(K // KB,),
        in_specs=[
            pl.BlockSpec((N_TOK,), lambda k: (0,)),
            pl.BlockSpec((N_TOK, D), lambda k: (0, 0)),
            pl.BlockSpec((K,), lambda k: (0,)),
            pl.BlockSpec((KB, D), lambda k: (k, 0)),
            pl.BlockSpec((KB,), lambda k: (k,)),
        ],
        out_specs=pl.BlockSpec((KB, D), lambda k: (k, 0)),
        out_shape=jax.ShapeDtypeStruct((K, D), jnp.float32),
        scratch_shapes=[pltpu.SMEM((1,), jnp.float32)],
    )(idx, f_bf16, cluster_size, ema_w, cluster_size)


def _gather_kernel(nw_hbm, idx_hbm, q_hbm, idx_v, rows_v, sem):
    wid = lax.axis_index("s") * NC + lax.axis_index("c")
    base = wid * GCHUNK
    pltpu.sync_copy(idx_hbm.at[pl.ds(base, GCHUNK)], idx_v)
    pltpu.async_copy(nw_hbm.at[idx_v], rows_v, sem).wait()
    pltpu.sync_copy(rows_v, q_hbm.at[pl.ds(base, GCHUNK)])


def _gather_rows(new_weight, idx):
    n = idx.shape[0]
    mesh = plsc.VectorSubcoreMesh(core_axis_name="c", subcore_axis_name="s")
    f = pl.kernel(
        _gather_kernel,
        out_type=jax.ShapeDtypeStruct((n, D), jnp.float32),
        mesh=mesh,
        scratch_types=[
            pltpu.VMEM((GCHUNK,), jnp.int32),
            pltpu.VMEM((GCHUNK, D), jnp.float32),
            pltpu.SemaphoreType.DMA,
        ],
    )
    return f(new_weight, idx)


def _loss_tr_body(q_ref, f_ref, out_ref, loss_ref, acc_ref):
    b = pl.program_id(0)
    q = q_ref[0]
    fl = f_ref[0]
    diff = q - fl
    part = jnp.sum(diff * diff)

    @pl.when(b == 0)
    def _():
        acc_ref[0] = 0.0

    acc_ref[0] += part
    out_ref[0] = q.T

    @pl.when(b == pl.num_programs(0) - 1)
    def _():
        loss_ref[...] = jnp.full((1, 1), acc_ref[0], jnp.float32)


def _loss_and_transpose(q_flat, flatten, batch, hw):
    out, loss = pl.pallas_call(
        _loss_tr_body,
        grid=(batch,),
        in_specs=[
            pl.BlockSpec((1, hw, D), lambda b: (b, 0, 0)),
            pl.BlockSpec((1, hw, D), lambda b: (b, 0, 0)),
        ],
        out_specs=[
            pl.BlockSpec((1, D, hw), lambda b: (b, 0, 0)),
            pl.BlockSpec((1, 1), lambda b: (0, 0)),
        ],
        out_shape=[
            jax.ShapeDtypeStruct((batch, D, hw), jnp.float32),
            jax.ShapeDtypeStruct((1, 1), jnp.float32),
        ],
        scratch_shapes=[pltpu.SMEM((1,), jnp.float32)],
    )(q_flat.reshape(batch, hw, D), flatten.reshape(batch, hw, D))
    return out, loss[0, 0]


def kernel(z_e_x, weight, ema_w, cluster_size):
    B, _, H, W = z_e_x.shape
    hw = H * W
    flatten = jnp.transpose(z_e_x, (0, 2, 3, 1)).reshape(-1, D)

    idx = _argmin_tokens(flatten, flatten.astype(jnp.bfloat16),
                         weight, weight.astype(jnp.bfloat16))
    new_weight = _segsum_ema(flatten.astype(jnp.float8_e4m3fn), idx,
                             ema_w, cluster_size)

    q_flat = _gather_rows(new_weight, idx)
    out, lsum = _loss_and_transpose(q_flat, flatten, B, hw)
    latent_loss = lsum / (N_TOK * D)
    return out.reshape(B, D, H, W), latent_loss


# speedup vs baseline: 1.0273x; 1.0273x over previous
import functools

import jax
import jax.numpy as jnp
from jax import lax
from jax.experimental import pallas as pl
from jax.experimental.pallas import tpu as pltpu
from jax.experimental.pallas import tpu_sc as plsc

K = 8192
D = 256
DECAY = 0.99
EPS = 1e-5

TM = 512
N_TOK = 8192
KB = 1024

NC, NS = 2, 16
GCHUNK = 256


def _argmin_body(f_ref, w_ref, idx_ref, wn2_ref):
    i = pl.program_id(0)

    @pl.when(i == 0)
    def _():
        w = w_ref[...]
        wn2_ref[...] = jnp.sum(w * w, axis=1, keepdims=True).reshape(1, K)

    f = f_ref[...]
    fn2 = jnp.sum(f * f, axis=1, keepdims=True)
    mm = lax.dot_general(
        f, w_ref[...], (((1,), (1,)), ((), ())),
        preferred_element_type=jnp.float32,
    )
    dist = (fn2 + wn2_ref[...]) - 2.0 * mm
    dmin = jnp.min(dist, axis=1, keepdims=True)
    ids = lax.broadcasted_iota(jnp.int32, dist.shape, 1)
    idx = jnp.min(jnp.where(dist == dmin, ids, K), axis=1)
    idx_ref[...] = idx.reshape(1, 1, TM)


def _argmin_tokens(flatten, weight):
    grid = (N_TOK // TM,)
    idx = pl.pallas_call(
        _argmin_body,
        grid=grid,
        in_specs=[
            pl.BlockSpec((TM, D), lambda i: (i, 0)),
            pl.BlockSpec((K, D), lambda i: (0, 0)),
        ],
        out_specs=pl.BlockSpec((1, 1, TM), lambda i: (i, 0, 0)),
        out_shape=jax.ShapeDtypeStruct((grid[0], 1, TM), jnp.int32),
        scratch_shapes=[pltpu.VMEM((1, K), jnp.float32)],
    )(flatten, weight)
    return idx.reshape(-1)


def _segsum_ema_body(idx_ref, fa_ref, csfull_ref, ema_ref, cs_ref, nw_ref,
                     n_ref):
    kb = pl.program_id(0)

    @pl.when(kb == 0)
    def _():
        n_ref[0] = (DECAY * jnp.sum(csfull_ref[...])
                    + (1.0 - DECAY) * N_TOK)

    idx_col = idx_ref[...].reshape(N_TOK, 1)
    ids = kb * KB + lax.broadcasted_iota(jnp.int32, (N_TOK, KB), 1)
    hit = idx_col == ids
    enc = jnp.where(hit, 1.0, 0.0).astype(jnp.float8_e4m3fn)
    emb = lax.dot_general(
        enc, fa_ref[...], (((0,), (0,)), ((), ())),
        preferred_element_type=jnp.float32,
    )
    counts = jnp.sum(jnp.where(hit, 1.0, 0.0), axis=0,
                     keepdims=True).reshape(KB, 1)
    cs = cs_ref[...].reshape(KB, 1) * DECAY + (1.0 - DECAY) * counts
    n = n_ref[0]
    csn = (cs + EPS) / (n + K * EPS) * n
    nw = (ema_ref[...] * DECAY + (1.0 - DECAY) * emb) / csn
    nw_ref[...] = nw.astype(jnp.bfloat16).astype(jnp.float32)


def _segsum_ema(f_bf16, idx, ema_w, cluster_size):
    return pl.pallas_call(
        _segsum_ema_body,
        grid=(K // KB,),
        in_specs=[
            pl.BlockSpec((N_TOK,), lambda k: (0,)),
            pl.BlockSpec((N_TOK, D), lambda k: (0, 0)),
            pl.BlockSpec((K,), lambda k: (0,)),
            pl.BlockSpec((KB, D), lambda k: (k, 0)),
            pl.BlockSpec((KB,), lambda k: (k,)),
        ],
        out_specs=pl.BlockSpec((KB, D), lambda k: (k, 0)),
        out_shape=jax.ShapeDtypeStruct((K, D), jnp.float32),
        scratch_shapes=[pltpu.SMEM((1,), jnp.float32)],
    )(idx, f_bf16, cluster_size, ema_w, cluster_size)


def _gather_kernel(nw_hbm, idx_hbm, q_hbm, idx_v, rows_v, sem):
    wid = lax.axis_index("s") * NC + lax.axis_index("c")
    base = wid * GCHUNK
    pltpu.sync_copy(idx_hbm.at[pl.ds(base, GCHUNK)], idx_v)
    pltpu.async_copy(nw_hbm.at[idx_v], rows_v, sem).wait()
    pltpu.sync_copy(rows_v, q_hbm.at[pl.ds(base, GCHUNK)])


def _gather_rows(new_weight, idx):
    n = idx.shape[0]
    mesh = plsc.VectorSubcoreMesh(core_axis_name="c", subcore_axis_name="s")
    f = pl.kernel(
        _gather_kernel,
        out_type=jax.ShapeDtypeStruct((n, D), jnp.float32),
        mesh=mesh,
        scratch_types=[
            pltpu.VMEM((GCHUNK,), jnp.int32),
            pltpu.VMEM((GCHUNK, D), jnp.float32),
            pltpu.SemaphoreType.DMA,
        ],
    )
    return f(new_weight, idx)


def _loss_tr_body(q_ref, f_ref, out_ref, loss_ref, acc_ref):
    b = pl.program_id(0)
    q = q_ref[0]
    fl = f_ref[0]
    diff = q - fl
    part = jnp.sum(diff * diff)

    @pl.when(b == 0)
    def _():
        acc_ref[0] = 0.0

    acc_ref[0] += part
    out_ref[0] = q.T

    @pl.when(b == pl.num_programs(0) - 1)
    def _():
        loss_ref[...] = jnp.full((1, 1), acc_ref[0], jnp.float32)


def _loss_and_transpose(q_flat, flatten, batch, hw):
    out, loss = pl.pallas_call(
        _loss_tr_body,
        grid=(batch,),
        in_specs=[
            pl.BlockSpec((1, hw, D), lambda b: (b, 0, 0)),
            pl.BlockSpec((1, hw, D), lambda b: (b, 0, 0)),
        ],
        out_specs=[
            pl.BlockSpec((1, D, hw), lambda b: (b, 0, 0)),
            pl.BlockSpec((1, 1), lambda b: (0, 0)),
        ],
        out_shape=[
            jax.ShapeDtypeStruct((batch, D, hw), jnp.float32),
            jax.ShapeDtypeStruct((1, 1), jnp.float32),
        ],
        scratch_shapes=[pltpu.SMEM((1,), jnp.float32)],
    )(q_flat.reshape(batch, hw, D), flatten.reshape(batch, hw, D))
    return out, loss[0, 0]


def kernel(z_e_x, weight, ema_w, cluster_size):
    B, _, H, W = z_e_x.shape
    hw = H * W
    flatten = jnp.transpose(z_e_x, (0, 2, 3, 1)).reshape(-1, D)

    idx = _argmin_tokens(flatten, weight)
    new_weight = _segsum_ema(flatten.astype(jnp.float8_e4m3fn), idx,
                             ema_w, cluster_size)

    q_flat = _gather_rows(new_weight, idx)
    out, lsum = _loss_and_transpose(q_flat, flatten, B, hw)
    latent_loss = lsum / (N_TOK * D)
    return out.reshape(B, D, H, W), latent_loss
